# baseline (speedup 1.0000x reference)
import jax
import jax.numpy as jnp
from jax.experimental import pallas as pl
from jax.experimental.pallas import tpu as pltpu


def _lane_flip(a):
    S, W = a.shape
    g = 128 if W % 128 == 0 else W
    idx = jax.lax.broadcasted_iota(jnp.int32, (S, g), 1)
    chunks = [
        jnp.take_along_axis(a[:, i * g:(i + 1) * g], (g - 1) - idx, axis=1)
        for i in range(W // g)
    ]
    return jnp.concatenate(chunks[::-1], axis=1) if len(chunks) > 1 else chunks[0]


def _sub_flip(a):
    H, W = a.shape
    g = H // 8
    a3 = a.reshape(g, 8, W)
    sidx = jax.lax.broadcasted_iota(jnp.int32, (g, 8, W), 1)
    within = jnp.take_along_axis(a3, 7 - sidx, axis=1)
    out = jnp.concatenate([within[g - 1 - i:g - i] for i in range(g)], axis=0)
    return out.reshape(H, W)


def _body(x_ref, o_ref, stage, sems):
    H, W = x_ref.shape[1], x_ref.shape[4]
    N = o_ref.shape[0] // 4
    C = pl.num_programs(1)
    n = pl.program_id(0)
    ch = pl.program_id(1)
    k = n * C + ch
    last = N * C - 1
    sl = jax.lax.rem(k, 2)

    a = x_ref[0, :, 0, 0, :]
    af = _lane_flip(a)
    odd = jax.lax.broadcasted_iota(jnp.int32, (H, 1), 0) % 2 == 1

    t = jnp.swapaxes(a, 0, 1)
    tf = _lane_flip(t)

    dsts = [
        o_ref.at[n, ch],
        o_ref.at[N + n, ch],
        o_ref.at[2 * N + n, ch],
        o_ref.at[3 * N + n, ch],
    ]

    def _mk(j, dst, slot):
        return pltpu.make_async_copy(stage.at[slot, j], dst, sems.at[slot, j])

    @pl.when(k >= 2)
    def _():
        for j in range(4):
            _mk(j, dsts[j], sl).wait()

    stage[sl, 0] = jnp.where(odd, af, a)
    stage[sl, 1] = _sub_flip(jnp.where(odd, a, af))
    stage[sl, 2] = jnp.where(odd, tf, t)
    stage[sl, 3] = _sub_flip(jnp.where(odd, t, tf))

    for j in range(4):
        _mk(j, dsts[j], sl).start()

    @pl.when(k == last)
    def _():
        for j in range(4):
            _mk(j, dsts[j], sl).wait()
            _mk(j, dsts[j], 1 - sl).wait()


def kernel(x_2d):
    N, H, W, C = x_2d.shape
    L = H * W

    xin = jnp.transpose(x_2d, (0, 1, 3, 2)).reshape(N, H, C, 1, W)

    out4 = pl.pallas_call(
        _body,
        grid=(N, C),
        in_specs=[
            pl.BlockSpec((1, H, 1, 1, W), lambda n, ch: (n, 0, ch, 0, 0)),
        ],
        out_specs=pl.BlockSpec(memory_space=pl.ANY),
        out_shape=jax.ShapeDtypeStruct((4 * N, C, H, W), x_2d.dtype),
        scratch_shapes=[
            pltpu.VMEM((2, 4, H, W), x_2d.dtype),
            pltpu.SemaphoreType.DMA((2, 4)),
        ],
    )(xin)

    return jnp.transpose(out4.reshape(4 * N, C, L), (0, 2, 1))


# speedup vs baseline: 2.4838x; 2.4838x over previous
import functools

import jax
import jax.numpy as jnp
from jax.experimental import pallas as pl


def _lane_flip3(a):
    M, S, W = a.shape
    g = 128 if W % 128 == 0 else W
    idx = jax.lax.broadcasted_iota(jnp.int32, (M, S, g), 2)
    chunks = [
        jnp.take_along_axis(a[:, :, i * g:(i + 1) * g], (g - 1) - idx, axis=2)
        for i in range(W // g)
    ]
    return jnp.concatenate(chunks[::-1], axis=2) if len(chunks) > 1 else chunks[0]


def _rev_major3(a):
    M, S, W = a.shape
    if M % 8 == 0 and M > 8:
        g = M // 8
        a4 = a.reshape(g, 8, S, W)
        inner = jnp.concatenate([a4[:, 7 - i:8 - i] for i in range(8)], axis=1)
        outer = jnp.concatenate([inner[g - 1 - i:g - i] for i in range(g)],
                                axis=0)
        return outer.reshape(M, S, W)
    return jnp.concatenate([a[M - 1 - i:M - i] for i in range(M)], axis=0)


def _hbody(x_ref, o_ref):
    H = x_ref.shape[1]
    di = pl.program_id(2)
    odd = jax.lax.broadcasted_iota(jnp.int32, (H, 1, 1), 0) % 2 == 1
    B = x_ref[0, :, 0]

    @pl.when(di == 0)
    def _():
        o_ref[0, 0] = jnp.where(odd, _lane_flip3(B), B)

    @pl.when(di == 1)
    def _():
        o_ref[0, 0] = _rev_major3(jnp.where(odd, B, _lane_flip3(B)))


def _vbody(x_ref, _prev_ref, o_ref):
    _hbody(x_ref, o_ref)


def kernel(x_2d):
    N, H, W, C = x_2d.shape
    L = H * W
    CG = C // 8

    xin = jnp.transpose(x_2d, (0, 1, 3, 2)).reshape(N, H, CG, 8, W)
    xt = jnp.transpose(xin, (0, 4, 2, 3, 1))

    out_shape = jax.ShapeDtypeStruct((4 * N, CG, H, 8, W), x_2d.dtype)

    h_out = pl.pallas_call(
        _hbody,
        grid=(N, CG, 2),
        in_specs=[
            pl.BlockSpec((1, H, 1, 8, W), lambda n, cg, di: (n, 0, cg, 0, 0)),
        ],
        out_specs=pl.BlockSpec(
            (1, 1, H, 8, W), lambda n, cg, di: (di * N + n, cg, 0, 0, 0)),
        out_shape=out_shape,
    )(xin)

    full = pl.pallas_call(
        _vbody,
        grid=(N, CG, 2),
        in_specs=[
            pl.BlockSpec((1, W, 1, 8, H), lambda n, cg, di: (n, 0, cg, 0, 0)),
            pl.BlockSpec(memory_space=pl.ANY),
        ],
        out_specs=pl.BlockSpec(
            (1, 1, W, 8, H),
            lambda n, cg, di: (2 * N + di * N + n, cg, 0, 0, 0)),
        out_shape=out_shape,
        input_output_aliases={1: 0},
    )(xt, h_out)

    return jnp.transpose(full, (0, 2, 4, 1, 3)).reshape(4 * N, L, C)


# speedup vs baseline: 2.4842x; 1.0002x over previous
import jax
import jax.numpy as jnp
from jax.experimental import pallas as pl


def _lane_flip3(a):
    M, S, W = a.shape
    g = 128 if W % 128 == 0 else W
    idx = jax.lax.broadcasted_iota(jnp.int32, (M, S, g), 2)
    chunks = [
        jnp.take_along_axis(a[:, :, i * g:(i + 1) * g], (g - 1) - idx, axis=2)
        for i in range(W // g)
    ]
    return jnp.concatenate(chunks[::-1], axis=2) if len(chunks) > 1 else chunks[0]


def _rev_major3(a):
    M, S, W = a.shape
    if M % 8 == 0 and M > 8:
        g = M // 8
        a4 = a.reshape(g, 8, S, W)
        inner = jnp.concatenate([a4[:, 7 - i:8 - i] for i in range(8)], axis=1)
        outer = jnp.concatenate([inner[g - 1 - i:g - i] for i in range(g)],
                                axis=0)
        return outer.reshape(M, S, W)
    return jnp.concatenate([a[M - 1 - i:M - i] for i in range(M)], axis=0)


def _hbody(x_ref, o_ref):
    H = x_ref.shape[1]
    di = pl.program_id(2)
    odd = jax.lax.broadcasted_iota(jnp.int32, (H, 1, 1), 0) % 2 == 1
    B = x_ref[0, :, 0]

    @pl.when(di == 0)
    def _():
        o_ref[0, 0] = jnp.where(odd, _lane_flip3(B), B)

    @pl.when(di == 1)
    def _():
        o_ref[0, 0] = _rev_major3(jnp.where(odd, B, _lane_flip3(B)))


def _vbody(x_ref, _prev_ref, o_ref):
    _hbody(x_ref, o_ref)


def kernel(x_2d):
    N, H, W, C = x_2d.shape
    L = H * W
    CG = C // 8

    xin = jnp.transpose(x_2d, (0, 1, 3, 2)).reshape(N, H, CG, 8, W)
    xt = jnp.transpose(xin, (0, 4, 2, 3, 1))

    out_shape = jax.ShapeDtypeStruct((4 * N, CG, H, 8, W), x_2d.dtype)

    h_out = pl.pallas_call(
        _hbody,
        grid=(N, CG, 2),
        in_specs=[
            pl.BlockSpec((1, H, 1, 8, W), lambda n, cg, di: (n, 0, cg, 0, 0)),
        ],
        out_specs=pl.BlockSpec(
            (1, 1, H, 8, W), lambda n, cg, di: (di * N + n, cg, 0, 0, 0)),
        out_shape=out_shape,
    )(xin)

    full = pl.pallas_call(
        _vbody,
        grid=(N, CG, 2),
        in_specs=[
            pl.BlockSpec((1, W, 1, 8, H), lambda n, cg, di: (n, 0, cg, 0, 0)),
            pl.BlockSpec(memory_space=pl.ANY),
        ],
        out_specs=pl.BlockSpec(
            (1, 1, W, 8, H),
            lambda n, cg, di: (2 * N + di * N + n, cg, 0, 0, 0)),
        out_shape=out_shape,
        input_output_aliases={1: 0},
    )(xt, h_out)

    return jnp.transpose(full, (0, 2, 4, 1, 3)).reshape(4 * N, L, C)
